# baseline (speedup 1.0000x reference)
import functools
import jax
import jax.numpy as jnp
from jax import lax
from jax.experimental import pallas as pl
from jax.experimental.pallas import tpu as pltpu
from jax.experimental.pallas import tpu_sc as plsc

N = 10000
E = 160000
L = 3
D_IN = 128
H = 8
D_OUT = 64
HD = H * D_OUT
SLOPE = 0.01

EB = 1280
NSTEPS = E // EB

NC = 2
NS = 16
LANES = 16
HH = H // NC

E_PAD = 163840
TCHUNK = E_PAD // NS
SUB = TCHUNK // 2
SGROUPS = SUB // LANES
NP = N + 112

EH_A = 80640
EH_B = E - EH_A
AGG_B = 80
CCOL = HD // NC
PCOL = 128
AGG_STRIPE = 640
ACC_ROWS = NS * AGG_STRIPE
WB = 624


def _enc_body(em_ref, v_ref, elT_ref, elmax_ref):
    em = em_ref[...]
    el = jnp.dot(em, v_ref[...], preferred_element_type=jnp.float32)
    elT_ref[...] = el.T
    cur = jnp.max(el, axis=0, keepdims=True)
    elmax_ref[...] = jnp.where(pl.program_id(0) == 0, cur,
                               jnp.maximum(elmax_ref[...], cur))


def _er_body(nf_ref, wr_ref, wenc_ref, attnl_ref, erT_ref, ermax_ref, v_ref):
    er = jnp.dot(nf_ref[...], wr_ref[...].T, preferred_element_type=jnp.float32)
    erT_ref[...] = er.T
    ermax_ref[...] = jnp.max(er, axis=0, keepdims=True)
    w = wenc_ref[...] * attnl_ref[...]
    cols = []
    for h in range(H):
        cols.append(jnp.sum(w[:, h * D_OUT:(h + 1) * D_OUT], axis=1, keepdims=True))
    v_ref[...] = jnp.concatenate(cols, axis=1)


def _msg_body(em_ref, aT_ref, wenc_ref, msg_ref):
    ef = jnp.dot(em_ref[...], wenc_ref[...], preferred_element_type=jnp.float32)
    a = aT_ref[...].T
    for h in range(H):
        msg_ref[h // 2, :, (h % 2) * D_OUT:(h % 2 + 1) * D_OUT] = (
            ef[:, h * D_OUT:(h + 1) * D_OUT] * a[:, h:h + 1])


_MESH = plsc.VectorSubcoreMesh(
    core_axis_name="c", subcore_axis_name="s", num_cores=NC, num_subcores=NS)
_SC_PARAMS = pltpu.CompilerParams(use_tc_tiling_on_sc=False,
                                  needs_layout_passes=False)


@functools.partial(
    pl.kernel,
    out_type=jax.ShapeDtypeStruct((NC, HH, E_PAD), jnp.float32),
    mesh=_MESH,
    scratch_types=[
        pltpu.VMEM((TCHUNK,), jnp.int32),
        pltpu.VMEM((HH, SUB), jnp.float32),
        pltpu.VMEM((HH, NP), jnp.float32),
        pltpu.VMEM((HH, NP), jnp.float32),
        pltpu.VMEM((HH, LANES), jnp.float32),
        pltpu.VMEM((HH,), jnp.int32),
        pltpu.VMEM_SHARED((HH, NP), jnp.float32),
    ],
    compiler_params=_SC_PARAMS,
)
def _sc_softmax(elT_hbm, dst_hbm, erT_hbm, cs_hbm, zeros4_hbm, rowidx_hbm,
                aT_hbm, dst_v, buf_v, er_v, part_v, c_v, rowidx_v, acc_sh):
    c = lax.axis_index("c")
    s = lax.axis_index("s")
    chunk = s * TCHUNK

    pltpu.sync_copy(dst_hbm.at[pl.ds(chunk, TCHUNK)], dst_v)
    pltpu.sync_copy(erT_hbm.at[c], er_v)
    pltpu.sync_copy(cs_hbm.at[c], c_v)
    pltpu.sync_copy(rowidx_hbm, rowidx_v)
    pltpu.sync_copy(zeros4_hbm, part_v)

    @pl.when(s == 0)
    def _():
        pltpu.sync_copy(zeros4_hbm, acc_sh)
    plsc.subcore_barrier()

    for sub in range(2):
        pltpu.sync_copy(
            elT_hbm.at[c, :, pl.ds(chunk + sub * SUB, SUB)], buf_v)

        def p1(g, carry):
            off = pl.multiple_of(g * LANES, LANES)
            idx = dst_v[pl.ds(sub * SUB + off, LANES)]
            for h in range(HH):
                hv = jnp.full((LANES,), h, jnp.int32)
                l = buf_v[h, pl.ds(off, LANES)]
                r = plsc.load_gather(er_v, [hv, idx])
                e = l + r
                e = jnp.where(e > 0, e, SLOPE * e)
                ex = jnp.exp(e - c_v[h, :])
                buf_v[h, pl.ds(off, LANES)] = ex
                plsc.addupdate_scatter(part_v, [hv, idx], ex)
            return carry

        lax.fori_loop(0, SGROUPS, p1, 0)
        pltpu.sync_copy(
            buf_v, aT_hbm.at[c, :, pl.ds(chunk + sub * SUB, SUB)])

    pltpu.sync_copy(part_v, acc_sh.at[rowidx_v], add=True)
    plsc.subcore_barrier()
    pltpu.sync_copy(acc_sh, er_v)

    for sub in range(2):
        pltpu.sync_copy(
            aT_hbm.at[c, :, pl.ds(chunk + sub * SUB, SUB)], buf_v)

        def p3(g, carry):
            off = pl.multiple_of(g * LANES, LANES)
            idx = dst_v[pl.ds(sub * SUB + off, LANES)]
            for h in range(HH):
                hv = jnp.full((LANES,), h, jnp.int32)
                d = plsc.load_gather(er_v, [hv, idx])
                buf_v[h, pl.ds(off, LANES)] = buf_v[h, pl.ds(off, LANES)] / d
            return carry

        lax.fori_loop(0, SGROUPS, p3, 0)
        pltpu.sync_copy(
            buf_v, aT_hbm.at[c, :, pl.ds(chunk + sub * SUB, SUB)])


def _make_sc_aggregate(eh, with_init):
  agg_t = eh // NS
  agg_nb = agg_t // AGG_B

  @functools.partial(
      pl.kernel,
      out_type=jax.ShapeDtypeStruct((N, HD), jnp.float32),
      mesh=_MESH,
      scratch_types=[
          pltpu.VMEM((agg_t,), jnp.int32),
          pltpu.VMEM((AGG_B, PCOL), jnp.float32),
          pltpu.VMEM((AGG_B, PCOL), jnp.float32),
          pltpu.VMEM((AGG_B, PCOL), jnp.float32),
          pltpu.VMEM((AGG_B,), jnp.int32),
          pltpu.VMEM((AGG_B,), jnp.int32),
          pltpu.SemaphoreType.DMA,
          pltpu.SemaphoreType.DMA,
          pltpu.VMEM_SHARED((ACC_ROWS, PCOL), jnp.float32),
      ],
      compiler_params=_SC_PARAMS,
  )
  def _sc_aggregate(msg_hbm, dst_hbm, *rest):
    if with_init:
        (init_hbm, out_hbm,
         dst_v, buf0, buf1, zbuf, idx0, idx1, sem0, sem1, acc_sh) = rest
    else:
        (out_hbm,
         dst_v, buf0, buf1, zbuf, idx0, idx1, sem0, sem1, acc_sh) = rest
    AGG_NB = agg_nb
    c = lax.axis_index("c")
    s = lax.axis_index("s")
    estart = s * agg_t
    bufs, idxs, sems = (buf0, buf1), (idx0, idx1), (sem0, sem1)

    pltpu.sync_copy(dst_hbm.at[pl.ds(estart, agg_t)], dst_v)

    zv = jnp.zeros((LANES,), jnp.float32)

    def zb(i, carry):
        r = i // (PCOL // LANES)
        k = lax.rem(i, PCOL // LANES)
        zbuf[r, pl.ds(pl.multiple_of(k * LANES, LANES), LANES)] = zv
        return carry

    lax.fori_loop(0, AGG_B * PCOL // LANES, zb, 0)

    for cp in range(CCOL // PCOL):
        colbase = c * CCOL + cp * PCOL
        cg = c * (CCOL // PCOL) + cp

        def _slab(b):
            boff = pl.multiple_of(b * AGG_B, 8)
            return msg_hbm.at[cg, pl.ds(estart + boff, AGG_B), :]

        if with_init:
            @pl.when(s < NS - 1)
            def _():
                pltpu.sync_copy(
                    init_hbm.at[pl.ds(s * AGG_STRIPE, AGG_STRIPE),
                                pl.ds(colbase, PCOL)],
                    acc_sh.at[pl.ds(s * AGG_STRIPE, AGG_STRIPE)])

            @pl.when(s == NS - 1)
            def _():
                pltpu.sync_copy(
                    init_hbm.at[pl.ds((NS - 1) * AGG_STRIPE,
                                      N - (NS - 1) * AGG_STRIPE),
                                pl.ds(colbase, PCOL)],
                    acc_sh.at[pl.ds((NS - 1) * AGG_STRIPE,
                                    N - (NS - 1) * AGG_STRIPE)])
        else:
            for z8 in range(AGG_STRIPE // AGG_B):
                pltpu.sync_copy(
                    zbuf, acc_sh.at[pl.ds(s * AGG_STRIPE + z8 * AGG_B, AGG_B)])
        plsc.subcore_barrier()

        pltpu.async_copy(_slab(0), buf0, sem0)
        pltpu.async_copy(_slab(1), buf1, sem1)

        def blk(b, carry):
            for slot in range(2):
                @pl.when(lax.rem(b, 2) == slot)
                def _():
                    buf, idxv, sem = bufs[slot], idxs[slot], sems[slot]

                    def grp(j, carry2):
                        joff = pl.multiple_of(j * LANES, LANES)
                        boff = pl.multiple_of(b * AGG_B, 8)
                        idxv[pl.ds(joff, LANES)] = dst_v[pl.ds(boff + joff, LANES)]
                        return carry2

                    lax.fori_loop(0, AGG_B // LANES, grp, 0)
                    pltpu.make_async_copy(_slab(b), buf, sem).wait()
                    pltpu.sync_copy(buf, acc_sh.at[idxv], add=True)

                    @pl.when(b + 2 < AGG_NB)
                    def _():
                        pltpu.async_copy(_slab(b + 2), buf, sem)
            return carry

        lax.fori_loop(0, AGG_NB, blk, 0)
        plsc.subcore_barrier()

        pltpu.sync_copy(
            acc_sh.at[pl.ds(s * WB, WB)],
            out_hbm.at[pl.ds(s * WB, WB), pl.ds(colbase, PCOL)])

        @pl.when(s == 0)
        def _():
            pltpu.sync_copy(
                acc_sh.at[pl.ds(NS * WB, N - NS * WB)],
                out_hbm.at[pl.ds(NS * WB, N - NS * WB), pl.ds(colbase, PCOL)])
        plsc.subcore_barrier()

  return _sc_aggregate


_SC_AGG_A = _make_sc_aggregate(EH_A, False)
_SC_AGG_B = _make_sc_aggregate(EH_B, True)


def _msg_call(em, aT, wenc, eh, offb):
    nsteps = eh // EB
    return pl.pallas_call(
        _msg_body,
        grid=(nsteps,),
        in_specs=[
            pl.BlockSpec((EB, D_IN), lambda i: (i + offb, 0)),
            pl.BlockSpec((H, EB), lambda i: (0, i + offb)),
            pl.BlockSpec((D_IN, HD), lambda i: (0, 0)),
        ],
        out_specs=pl.BlockSpec((HD // PCOL, EB, PCOL), lambda i: (0, i, 0)),
        out_shape=jax.ShapeDtypeStruct((HD // PCOL, eh, PCOL), jnp.float32),
    )(em, aT, wenc)


def kernel(node_feat, edge_feat, edge_index, W_enc, attn_l, W_r):
    dst = edge_index[1]
    attnl_flat = attn_l.reshape(1, HD)
    em = jnp.mean(edge_feat, axis=1)
    W_enc_l = W_enc

    erT, ermax, V = pl.pallas_call(
        _er_body,
        out_shape=[
            jax.ShapeDtypeStruct((H, N), jnp.float32),
            jax.ShapeDtypeStruct((1, H), jnp.float32),
            jax.ShapeDtypeStruct((D_IN, H), jnp.float32),
        ],
    )(node_feat, W_r, W_enc, attnl_flat)

    elT, elmax = pl.pallas_call(
        _enc_body,
        grid=(NSTEPS,),
        in_specs=[
            pl.BlockSpec((EB, D_IN), lambda i: (i, 0)),
            pl.BlockSpec((D_IN, H), lambda i: (0, 0)),
        ],
        out_specs=[
            pl.BlockSpec((H, EB), lambda i: (0, i)),
            pl.BlockSpec((1, H), lambda i: (0, 0)),
        ],
        out_shape=[
            jax.ShapeDtypeStruct((H, E), jnp.float32),
            jax.ShapeDtypeStruct((1, H), jnp.float32),
        ],
    )(em, V)

    C = jax.nn.relu(elmax[0] + ermax[0])

    elT_pad = jnp.pad(elT, ((0, 0), (0, E_PAD - E))).reshape(NC, HH, E_PAD)
    dst_pad = jnp.concatenate([dst, jnp.full((E_PAD - E,), N, jnp.int32)])
    erT_pad = jnp.pad(erT, ((0, 0), (0, NP - N))).reshape(NC, HH, NP)
    cs = jnp.broadcast_to(C[:, None], (H, LANES)).reshape(NC, HH, LANES)
    zeros4 = jnp.zeros((HH, NP), jnp.float32)
    rowidx = jnp.arange(HH, dtype=jnp.int32)

    aT_full = _sc_softmax(elT_pad, dst_pad, erT_pad, cs, zeros4, rowidx)
    aT = aT_full.reshape(H, E_PAD)[:, :E]

    msg_a = _msg_call(em, aT, W_enc_l, EH_A, 0)
    out_a = _SC_AGG_A(msg_a, dst[:EH_A])
    msg_b = _msg_call(em, aT, W_enc_l, EH_B, EH_A // EB)
    out = _SC_AGG_B(msg_b, dst[EH_A:], out_a)
    return out.reshape(N, H, D_OUT)


# speedup vs baseline: 1.0640x; 1.0640x over previous
import functools
import jax
import jax.numpy as jnp
from jax import lax
from jax.experimental import pallas as pl
from jax.experimental.pallas import tpu as pltpu
from jax.experimental.pallas import tpu_sc as plsc

N = 10000
E = 160000
L = 3
D_IN = 128
H = 8
D_OUT = 64
HD = H * D_OUT
SLOPE = 0.01

EB = 1280
NSTEPS = E // EB

NC = 2
NS = 16
LANES = 16
HH = H // NC

E_PAD = 163840
TCHUNK = E_PAD // NS
SUB = TCHUNK // 2
SGROUPS = SUB // LANES
NP = N + 112

EH_A = 80640
EH_B = E - EH_A
AGG_B = 80
CCOL = HD // NC
PCOL = 128
AGG_STRIPE = 640
ACC_ROWS = NS * AGG_STRIPE
WB = 624


def _enc_body(em_ref, v_ref, elT_ref, elmax_ref):
    em = em_ref[...]
    el = jnp.dot(em, v_ref[...], preferred_element_type=jnp.float32)
    elT_ref[...] = el.T
    cur = jnp.max(el, axis=0, keepdims=True)
    elmax_ref[...] = jnp.where(pl.program_id(0) == 0, cur,
                               jnp.maximum(elmax_ref[...], cur))


def _er_body(nf_ref, wr_ref, wenc_ref, attnl_ref, erT_ref, ermax_ref, v_ref):
    er = jnp.dot(nf_ref[...], wr_ref[...].T, preferred_element_type=jnp.float32)
    erT_ref[...] = er.T
    ermax_ref[...] = jnp.max(er, axis=0, keepdims=True)
    w = wenc_ref[...] * attnl_ref[...]
    cols = []
    for h in range(H):
        cols.append(jnp.sum(w[:, h * D_OUT:(h + 1) * D_OUT], axis=1, keepdims=True))
    v_ref[...] = jnp.concatenate(cols, axis=1)


def _msg_body(em_ref, aT_ref, wenc_ref, msg_ref):
    ef = jnp.dot(em_ref[...], wenc_ref[...], preferred_element_type=jnp.float32)
    a = aT_ref[...].T
    for h in range(H):
        msg_ref[h // 2, :, (h % 2) * D_OUT:(h % 2 + 1) * D_OUT] = (
            ef[:, h * D_OUT:(h + 1) * D_OUT] * a[:, h:h + 1])


_MESH = plsc.VectorSubcoreMesh(
    core_axis_name="c", subcore_axis_name="s", num_cores=NC, num_subcores=NS)
_SC_PARAMS = pltpu.CompilerParams(use_tc_tiling_on_sc=False,
                                  needs_layout_passes=False)


@functools.partial(
    pl.kernel,
    out_type=jax.ShapeDtypeStruct((NC, HH, E_PAD), jnp.float32),
    mesh=_MESH,
    scratch_types=[
        pltpu.VMEM((TCHUNK,), jnp.int32),
        pltpu.VMEM((HH, SUB), jnp.float32),
        pltpu.VMEM((HH, NP), jnp.float32),
        pltpu.VMEM((HH, NP), jnp.float32),
        pltpu.VMEM((HH, LANES), jnp.float32),
        pltpu.VMEM((HH,), jnp.int32),
        pltpu.VMEM_SHARED((HH, NP), jnp.float32),
    ],
    compiler_params=_SC_PARAMS,
)
def _sc_softmax(elT_hbm, dst_hbm, erT_hbm, cs_hbm, zeros4_hbm, rowidx_hbm,
                aT_hbm, dst_v, buf_v, er_v, part_v, c_v, rowidx_v, acc_sh):
    c = lax.axis_index("c")
    s = lax.axis_index("s")
    chunk = s * TCHUNK

    pltpu.sync_copy(dst_hbm.at[pl.ds(chunk, TCHUNK)], dst_v)
    pltpu.sync_copy(erT_hbm.at[c], er_v)
    pltpu.sync_copy(cs_hbm.at[c], c_v)
    pltpu.sync_copy(rowidx_hbm, rowidx_v)
    pltpu.sync_copy(zeros4_hbm, part_v)

    @pl.when(s == 0)
    def _():
        pltpu.sync_copy(zeros4_hbm, acc_sh)
    plsc.subcore_barrier()

    for sub in range(2):
        pltpu.sync_copy(
            elT_hbm.at[c, :, pl.ds(chunk + sub * SUB, SUB)], buf_v)

        def p1(g, carry):
            off = pl.multiple_of(g * LANES, LANES)
            idx = dst_v[pl.ds(sub * SUB + off, LANES)]
            for h in range(HH):
                hv = jnp.full((LANES,), h, jnp.int32)
                l = buf_v[h, pl.ds(off, LANES)]
                r = plsc.load_gather(er_v, [hv, idx])
                e = l + r
                e = jnp.where(e > 0, e, SLOPE * e)
                ex = jnp.exp(e - c_v[h, :])
                buf_v[h, pl.ds(off, LANES)] = ex
                plsc.addupdate_scatter(part_v, [hv, idx], ex)
            return carry

        lax.fori_loop(0, SGROUPS, p1, 0)
        pltpu.sync_copy(
            buf_v, aT_hbm.at[c, :, pl.ds(chunk + sub * SUB, SUB)])

    pltpu.sync_copy(part_v, acc_sh.at[rowidx_v], add=True)
    plsc.subcore_barrier()
    pltpu.sync_copy(acc_sh, er_v)

    for sub in range(2):
        pltpu.sync_copy(
            aT_hbm.at[c, :, pl.ds(chunk + sub * SUB, SUB)], buf_v)

        def p3(g, carry):
            off = pl.multiple_of(g * LANES, LANES)
            idx = dst_v[pl.ds(sub * SUB + off, LANES)]
            for h in range(HH):
                hv = jnp.full((LANES,), h, jnp.int32)
                d = plsc.load_gather(er_v, [hv, idx])
                buf_v[h, pl.ds(off, LANES)] = buf_v[h, pl.ds(off, LANES)] / d
            return carry

        lax.fori_loop(0, SGROUPS, p3, 0)
        pltpu.sync_copy(
            buf_v, aT_hbm.at[c, :, pl.ds(chunk + sub * SUB, SUB)])


def _make_sc_aggregate(eh, with_init):
  agg_t = eh // NS
  agg_nb = agg_t // AGG_B

  @functools.partial(
      pl.kernel,
      out_type=jax.ShapeDtypeStruct((N, HD), jnp.float32),
      mesh=_MESH,
      scratch_types=[
          pltpu.VMEM((agg_t,), jnp.int32),
          pltpu.VMEM((AGG_B, PCOL), jnp.float32),
          pltpu.VMEM((AGG_B, PCOL), jnp.float32),
          pltpu.VMEM((AGG_B, PCOL), jnp.float32),
          pltpu.VMEM((AGG_B,), jnp.int32),
          pltpu.VMEM((AGG_B,), jnp.int32),
          pltpu.SemaphoreType.DMA,
          pltpu.SemaphoreType.DMA,
          pltpu.VMEM_SHARED((ACC_ROWS, PCOL), jnp.float32),
      ],
      compiler_params=_SC_PARAMS,
  )
  def _sc_aggregate(msg_hbm, dst_hbm, *rest):
    if with_init:
        (init_hbm, out_hbm,
         dst_v, buf0, buf1, zbuf, idx0, idx1, sem0, sem1, acc_sh) = rest
    else:
        (out_hbm,
         dst_v, buf0, buf1, zbuf, idx0, idx1, sem0, sem1, acc_sh) = rest
    AGG_NB = agg_nb
    c = lax.axis_index("c")
    s = lax.axis_index("s")
    estart = s * agg_t
    bufs, idxs, sems = (buf0, buf1), (idx0, idx1), (sem0, sem1)

    pltpu.sync_copy(dst_hbm.at[pl.ds(estart, agg_t)], dst_v)

    zv = jnp.zeros((LANES,), jnp.float32)

    def zb(i, carry):
        r = i // (PCOL // LANES)
        k = lax.rem(i, PCOL // LANES)
        zbuf[r, pl.ds(pl.multiple_of(k * LANES, LANES), LANES)] = zv
        return carry

    lax.fori_loop(0, AGG_B * PCOL // LANES, zb, 0)

    for cp in range(CCOL // PCOL):
        colbase = c * CCOL + cp * PCOL
        cg = c * (CCOL // PCOL) + cp

        def _slab(b):
            boff = pl.multiple_of(b * AGG_B, 8)
            return msg_hbm.at[cg, pl.ds(estart + boff, AGG_B), :]

        if with_init:
            @pl.when(s < NS - 1)
            def _():
                pltpu.sync_copy(
                    init_hbm.at[pl.ds(s * AGG_STRIPE, AGG_STRIPE),
                                pl.ds(colbase, PCOL)],
                    acc_sh.at[pl.ds(s * AGG_STRIPE, AGG_STRIPE)])

            @pl.when(s == NS - 1)
            def _():
                pltpu.sync_copy(
                    init_hbm.at[pl.ds((NS - 1) * AGG_STRIPE,
                                      N - (NS - 1) * AGG_STRIPE),
                                pl.ds(colbase, PCOL)],
                    acc_sh.at[pl.ds((NS - 1) * AGG_STRIPE,
                                    N - (NS - 1) * AGG_STRIPE)])
        else:
            for z8 in range(AGG_STRIPE // AGG_B):
                pltpu.sync_copy(
                    zbuf, acc_sh.at[pl.ds(s * AGG_STRIPE + z8 * AGG_B, AGG_B)])
        plsc.subcore_barrier()

        pltpu.async_copy(_slab(0), buf0, sem0)
        pltpu.async_copy(_slab(1), buf1, sem1)

        def blk(b, carry):
            for slot in range(2):
                @pl.when(lax.rem(b, 2) == slot)
                def _():
                    buf, idxv, sem = bufs[slot], idxs[slot], sems[slot]

                    def grp(j, carry2):
                        joff = pl.multiple_of(j * LANES, LANES)
                        boff = pl.multiple_of(b * AGG_B, 8)
                        idxv[pl.ds(joff, LANES)] = dst_v[pl.ds(boff + joff, LANES)]
                        return carry2

                    lax.fori_loop(0, AGG_B // LANES, grp, 0)
                    pltpu.make_async_copy(_slab(b), buf, sem).wait()
                    pltpu.sync_copy(buf, acc_sh.at[idxv], add=True)

                    @pl.when(b + 2 < AGG_NB)
                    def _():
                        pltpu.async_copy(_slab(b + 2), buf, sem)
            return carry

        lax.fori_loop(0, AGG_NB, blk, 0)
        plsc.subcore_barrier()

        pltpu.sync_copy(
            acc_sh.at[pl.ds(s * WB, WB)],
            out_hbm.at[pl.ds(s * WB, WB), pl.ds(colbase, PCOL)])

        @pl.when(s == 0)
        def _():
            pltpu.sync_copy(
                acc_sh.at[pl.ds(NS * WB, N - NS * WB)],
                out_hbm.at[pl.ds(NS * WB, N - NS * WB), pl.ds(colbase, PCOL)])
        plsc.subcore_barrier()

  return _sc_aggregate


_SC_AGG_A = _make_sc_aggregate(EH_A, False)
_SC_AGG_B = _make_sc_aggregate(EH_B, True)


def _msg_call(em, aT, wenc, eh, offb):
    nsteps = eh // EB
    return pl.pallas_call(
        _msg_body,
        grid=(nsteps,),
        in_specs=[
            pl.BlockSpec((EB, D_IN), lambda i: (i + offb, 0)),
            pl.BlockSpec((H, EB), lambda i: (0, i + offb)),
            pl.BlockSpec((D_IN, HD), lambda i: (0, 0)),
        ],
        out_specs=pl.BlockSpec((HD // PCOL, EB, PCOL), lambda i: (0, i, 0)),
        out_shape=jax.ShapeDtypeStruct((HD // PCOL, eh, PCOL), jnp.float32),
    )(em, aT, wenc)


def kernel(node_feat, edge_feat, edge_index, W_enc, attn_l, W_r):
    dst = edge_index[1]
    attnl_flat = attn_l.reshape(1, HD)
    em = (edge_feat[:, 0, :] + edge_feat[:, 1, :] + edge_feat[:, 2, :]) * (
        jnp.float32(1.0 / L))
    W_enc_l = W_enc

    erT, ermax, V = pl.pallas_call(
        _er_body,
        out_shape=[
            jax.ShapeDtypeStruct((H, N), jnp.float32),
            jax.ShapeDtypeStruct((1, H), jnp.float32),
            jax.ShapeDtypeStruct((D_IN, H), jnp.float32),
        ],
    )(node_feat, W_r, W_enc, attnl_flat)

    elT, elmax = pl.pallas_call(
        _enc_body,
        grid=(NSTEPS,),
        in_specs=[
            pl.BlockSpec((EB, D_IN), lambda i: (i, 0)),
            pl.BlockSpec((D_IN, H), lambda i: (0, 0)),
        ],
        out_specs=[
            pl.BlockSpec((H, EB), lambda i: (0, i)),
            pl.BlockSpec((1, H), lambda i: (0, 0)),
        ],
        out_shape=[
            jax.ShapeDtypeStruct((H, E), jnp.float32),
            jax.ShapeDtypeStruct((1, H), jnp.float32),
        ],
    )(em, V)

    C = jax.nn.relu(elmax[0] + ermax[0])

    elT_pad = jnp.pad(elT, ((0, 0), (0, E_PAD - E))).reshape(NC, HH, E_PAD)
    dst_pad = jnp.concatenate([dst, jnp.full((E_PAD - E,), N, jnp.int32)])
    erT_pad = jnp.pad(erT, ((0, 0), (0, NP - N))).reshape(NC, HH, NP)
    cs = jnp.broadcast_to(C[:, None], (H, LANES)).reshape(NC, HH, LANES)
    zeros4 = jnp.zeros((HH, NP), jnp.float32)
    rowidx = jnp.arange(HH, dtype=jnp.int32)

    aT_full = _sc_softmax(elT_pad, dst_pad, erT_pad, cs, zeros4, rowidx)
    aT = aT_full.reshape(H, E_PAD)[:, :E]

    msg_a = _msg_call(em, aT, W_enc_l, EH_A, 0)
    out_a = _SC_AGG_A(msg_a, dst[:EH_A])
    msg_b = _msg_call(em, aT, W_enc_l, EH_B, EH_A // EB)
    out = _SC_AGG_B(msg_b, dst[EH_A:], out_a)
    return out.reshape(N, H, D_OUT)


# speedup vs baseline: 1.1001x; 1.0339x over previous
import functools
import jax
import jax.numpy as jnp
from jax import lax
from jax.experimental import pallas as pl
from jax.experimental.pallas import tpu as pltpu
from jax.experimental.pallas import tpu_sc as plsc

N = 10000
E = 160000
L = 3
D_IN = 128
H = 8
D_OUT = 64
HD = H * D_OUT
SLOPE = 0.01

EB = 1280
NSTEPS = E // EB

NC = 2
NS = 16
LANES = 16
HH = H // NC

E_PAD = 163840
TCHUNK = E_PAD // NS
SUB = TCHUNK // 2
SGROUPS = SUB // LANES
NP = N + 112

EH_A = 80640
EH_B = E - EH_A
AGG_B = 80
CCOL = HD // NC
PCOL = 128
AGG_STRIPE = 626
ACC_ROWS = NS * AGG_STRIPE
ZROWS = 48
WB = 624


def _enc_body(em_ref, v_ref, elT_ref, elmax_ref):
    em = em_ref[...]
    el = jnp.dot(em, v_ref[...], preferred_element_type=jnp.float32)
    elT_ref[...] = el.T
    cur = jnp.max(el, axis=0, keepdims=True)
    elmax_ref[...] = jnp.where(pl.program_id(0) == 0, cur,
                               jnp.maximum(elmax_ref[...], cur))


def _er_body(nf_ref, wr_ref, wenc_ref, attnl_ref, erT_ref, ermax_ref, v_ref):
    er = jnp.dot(nf_ref[...], wr_ref[...].T, preferred_element_type=jnp.float32)
    erT_ref[...] = er.T
    ermax_ref[...] = jnp.max(er, axis=0, keepdims=True)
    w = wenc_ref[...] * attnl_ref[...]
    cols = []
    for h in range(H):
        cols.append(jnp.sum(w[:, h * D_OUT:(h + 1) * D_OUT], axis=1, keepdims=True))
    v_ref[...] = jnp.concatenate(cols, axis=1)


def _msg_body(em_ref, aT_ref, wenc_ref, msg_ref):
    ef = jnp.dot(em_ref[...], wenc_ref[...], preferred_element_type=jnp.float32)
    a = aT_ref[...].T
    for h in range(H):
        msg_ref[h // 2, :, (h % 2) * D_OUT:(h % 2 + 1) * D_OUT] = (
            ef[:, h * D_OUT:(h + 1) * D_OUT] * a[:, h:h + 1])


_MESH = plsc.VectorSubcoreMesh(
    core_axis_name="c", subcore_axis_name="s", num_cores=NC, num_subcores=NS)
_SC_PARAMS = pltpu.CompilerParams(use_tc_tiling_on_sc=False,
                                  needs_layout_passes=False)


@functools.partial(
    pl.kernel,
    out_type=jax.ShapeDtypeStruct((NC, HH, E_PAD), jnp.float32),
    mesh=_MESH,
    scratch_types=[
        pltpu.VMEM((TCHUNK,), jnp.int32),
        pltpu.VMEM((HH, SUB), jnp.float32),
        pltpu.VMEM((HH, NP), jnp.float32),
        pltpu.VMEM((HH, NP), jnp.float32),
        pltpu.VMEM((HH, LANES), jnp.float32),
        pltpu.VMEM((HH,), jnp.int32),
        pltpu.VMEM_SHARED((HH, NP), jnp.float32),
    ],
    compiler_params=_SC_PARAMS,
)
def _sc_softmax(elT_hbm, dst_hbm, erT_hbm, cs_hbm, zeros4_hbm, rowidx_hbm,
                aT_hbm, dst_v, buf_v, er_v, part_v, c_v, rowidx_v, acc_sh):
    c = lax.axis_index("c")
    s = lax.axis_index("s")
    chunk = s * TCHUNK

    pltpu.sync_copy(dst_hbm.at[pl.ds(chunk, TCHUNK)], dst_v)
    pltpu.sync_copy(erT_hbm.at[c], er_v)
    pltpu.sync_copy(cs_hbm.at[c], c_v)
    pltpu.sync_copy(rowidx_hbm, rowidx_v)
    pltpu.sync_copy(zeros4_hbm, part_v)

    @pl.when(s == 0)
    def _():
        pltpu.sync_copy(zeros4_hbm, acc_sh)
    plsc.subcore_barrier()

    for sub in range(2):
        pltpu.sync_copy(
            elT_hbm.at[c, :, pl.ds(chunk + sub * SUB, SUB)], buf_v)

        def p1(g, carry):
            off = pl.multiple_of(g * LANES, LANES)
            idx = dst_v[pl.ds(sub * SUB + off, LANES)]
            for h in range(HH):
                hv = jnp.full((LANES,), h, jnp.int32)
                l = buf_v[h, pl.ds(off, LANES)]
                r = plsc.load_gather(er_v, [hv, idx])
                e = l + r
                e = jnp.where(e > 0, e, SLOPE * e)
                ex = jnp.exp(e - c_v[h, :])
                buf_v[h, pl.ds(off, LANES)] = ex
                plsc.addupdate_scatter(part_v, [hv, idx], ex)
            return carry

        lax.fori_loop(0, SGROUPS, p1, 0)
        pltpu.sync_copy(
            buf_v, aT_hbm.at[c, :, pl.ds(chunk + sub * SUB, SUB)])

    pltpu.sync_copy(part_v, acc_sh.at[rowidx_v], add=True)
    plsc.subcore_barrier()
    pltpu.sync_copy(acc_sh, er_v)

    for sub in range(2):
        pltpu.sync_copy(
            aT_hbm.at[c, :, pl.ds(chunk + sub * SUB, SUB)], buf_v)

        def p3(g, carry):
            off = pl.multiple_of(g * LANES, LANES)
            idx = dst_v[pl.ds(sub * SUB + off, LANES)]
            for h in range(HH):
                hv = jnp.full((LANES,), h, jnp.int32)
                d = plsc.load_gather(er_v, [hv, idx])
                buf_v[h, pl.ds(off, LANES)] = buf_v[h, pl.ds(off, LANES)] / d
            return carry

        lax.fori_loop(0, SGROUPS, p3, 0)
        pltpu.sync_copy(
            buf_v, aT_hbm.at[c, :, pl.ds(chunk + sub * SUB, SUB)])


def _make_sc_aggregate(eh, with_init):
  agg_t = eh // NS
  agg_nb = agg_t // AGG_B

  @functools.partial(
      pl.kernel,
      out_type=jax.ShapeDtypeStruct((N, HD), jnp.float32),
      mesh=_MESH,
      scratch_types=(
          [pltpu.VMEM((agg_t,), jnp.int32)]
          + [pltpu.VMEM((AGG_B, PCOL), jnp.float32) for _ in range(3)]
          + [pltpu.VMEM((ZROWS, PCOL), jnp.float32)]
          + [pltpu.VMEM((AGG_B,), jnp.int32) for _ in range(3)]
          + [pltpu.SemaphoreType.DMA for _ in range(6)]
          + [pltpu.VMEM_SHARED((ACC_ROWS, PCOL), jnp.float32)]
      ),
      compiler_params=_SC_PARAMS,
  )
  def _sc_aggregate(msg_hbm, dst_hbm, *rest):
    if with_init:
        init_hbm, out_hbm = rest[0], rest[1]
        rest = rest[2:]
    else:
        out_hbm = rest[0]
        rest = rest[1:]
    (dst_v, b0, b1, b2, zbuf, i0, i1, i2,
     g0, g1, g2, s0, s1, s2, acc_sh) = rest
    AGG_NB = agg_nb
    c = lax.axis_index("c")
    s = lax.axis_index("s")
    estart = s * agg_t
    bufs, idxs = (b0, b1, b2), (i0, i1, i2)
    gsems, ssems = (g0, g1, g2), (s0, s1, s2)

    pltpu.sync_copy(dst_hbm.at[pl.ds(estart, agg_t)], dst_v)

    zv = jnp.zeros((LANES,), jnp.float32)

    def zb(i, carry):
        r = i // (PCOL // LANES)
        k = lax.rem(i, PCOL // LANES)
        zbuf[r, pl.ds(pl.multiple_of(k * LANES, LANES), LANES)] = zv
        return carry

    lax.fori_loop(0, ZROWS * PCOL // LANES, zb, 0)

    for cp in range(CCOL // PCOL):
        colbase = c * CCOL + cp * PCOL
        cg = c * (CCOL // PCOL) + cp

        def _slab(b):
            boff = pl.multiple_of(b * AGG_B, 8)
            return msg_hbm.at[cg, pl.ds(estart + boff, AGG_B), :]

        if with_init:
            @pl.when(s < NS - 1)
            def _():
                pltpu.sync_copy(
                    init_hbm.at[pl.ds(s * AGG_STRIPE, AGG_STRIPE),
                                pl.ds(colbase, PCOL)],
                    acc_sh.at[pl.ds(s * AGG_STRIPE, AGG_STRIPE)])

            @pl.when(s == NS - 1)
            def _():
                pltpu.sync_copy(
                    init_hbm.at[pl.ds((NS - 1) * AGG_STRIPE,
                                      N - (NS - 1) * AGG_STRIPE),
                                pl.ds(colbase, PCOL)],
                    acc_sh.at[pl.ds((NS - 1) * AGG_STRIPE,
                                    N - (NS - 1) * AGG_STRIPE)])
        else:
            for z8 in range(AGG_STRIPE // ZROWS):
                pltpu.sync_copy(
                    zbuf, acc_sh.at[pl.ds(s * AGG_STRIPE + z8 * ZROWS, ZROWS)])
            pltpu.sync_copy(
                zbuf.at[pl.ds(0, AGG_STRIPE - ZROWS * (AGG_STRIPE // ZROWS))],
                acc_sh.at[pl.ds(
                    s * AGG_STRIPE + ZROWS * (AGG_STRIPE // ZROWS),
                    AGG_STRIPE - ZROWS * (AGG_STRIPE // ZROWS))])
        plsc.subcore_barrier()

        pltpu.async_copy(_slab(0), bufs[0], gsems[0])
        pltpu.async_copy(_slab(1), bufs[1], gsems[1])

        def blk(b, carry):
            for slot in range(3):
                @pl.when(lax.rem(b, 3) == slot)
                def _():
                    buf, idxv = bufs[slot], idxs[slot]

                    def grp(j, carry2):
                        joff = pl.multiple_of(j * LANES, LANES)
                        boff = pl.multiple_of(b * AGG_B, 8)
                        idxv[pl.ds(joff, LANES)] = dst_v[pl.ds(boff + joff, LANES)]
                        return carry2

                    lax.fori_loop(0, AGG_B // LANES, grp, 0)
                    pltpu.make_async_copy(_slab(b), buf, gsems[slot]).wait()
                    pltpu.async_copy(buf, acc_sh.at[idxv], ssems[slot], add=True)

                    pre = (slot + 2) % 3

                    @pl.when(b + 2 < AGG_NB)
                    def _():
                        @pl.when(b >= 1)
                        def _():
                            pltpu.make_async_copy(
                                bufs[pre], acc_sh.at[idxs[pre]], ssems[pre]).wait()
                        pltpu.async_copy(_slab(b + 2), bufs[pre], gsems[pre])
            return carry

        lax.fori_loop(0, AGG_NB, blk, 0)

        for k in range(max(0, agg_nb - 3), agg_nb):
            slot = k % 3
            pltpu.make_async_copy(
                bufs[slot], acc_sh.at[idxs[slot]], ssems[slot]).wait()
        plsc.subcore_barrier()

        pltpu.sync_copy(
            acc_sh.at[pl.ds(s * WB, WB)],
            out_hbm.at[pl.ds(s * WB, WB), pl.ds(colbase, PCOL)])

        @pl.when(s == 0)
        def _():
            pltpu.sync_copy(
                acc_sh.at[pl.ds(NS * WB, N - NS * WB)],
                out_hbm.at[pl.ds(NS * WB, N - NS * WB), pl.ds(colbase, PCOL)])
        plsc.subcore_barrier()

  return _sc_aggregate


_SC_AGG_A = _make_sc_aggregate(EH_A, False)
_SC_AGG_B = _make_sc_aggregate(EH_B, True)


def _msg_call(em, aT, wenc, eh, offb):
    nsteps = eh // EB
    return pl.pallas_call(
        _msg_body,
        grid=(nsteps,),
        in_specs=[
            pl.BlockSpec((EB, D_IN), lambda i: (i + offb, 0)),
            pl.BlockSpec((H, EB), lambda i: (0, i + offb)),
            pl.BlockSpec((D_IN, HD), lambda i: (0, 0)),
        ],
        out_specs=pl.BlockSpec((HD // PCOL, EB, PCOL), lambda i: (0, i, 0)),
        out_shape=jax.ShapeDtypeStruct((HD // PCOL, eh, PCOL), jnp.float32),
    )(em, aT, wenc)


def kernel(node_feat, edge_feat, edge_index, W_enc, attn_l, W_r):
    dst = edge_index[1]
    attnl_flat = attn_l.reshape(1, HD)
    em = (edge_feat[:, 0, :] + edge_feat[:, 1, :] + edge_feat[:, 2, :]) * (
        jnp.float32(1.0 / L))
    W_enc_l = W_enc

    erT, ermax, V = pl.pallas_call(
        _er_body,
        out_shape=[
            jax.ShapeDtypeStruct((H, N), jnp.float32),
            jax.ShapeDtypeStruct((1, H), jnp.float32),
            jax.ShapeDtypeStruct((D_IN, H), jnp.float32),
        ],
    )(node_feat, W_r, W_enc, attnl_flat)

    elT, elmax = pl.pallas_call(
        _enc_body,
        grid=(NSTEPS,),
        in_specs=[
            pl.BlockSpec((EB, D_IN), lambda i: (i, 0)),
            pl.BlockSpec((D_IN, H), lambda i: (0, 0)),
        ],
        out_specs=[
            pl.BlockSpec((H, EB), lambda i: (0, i)),
            pl.BlockSpec((1, H), lambda i: (0, 0)),
        ],
        out_shape=[
            jax.ShapeDtypeStruct((H, E), jnp.float32),
            jax.ShapeDtypeStruct((1, H), jnp.float32),
        ],
    )(em, V)

    C = jax.nn.relu(elmax[0] + ermax[0])

    elT_pad = jnp.pad(elT, ((0, 0), (0, E_PAD - E))).reshape(NC, HH, E_PAD)
    dst_pad = jnp.concatenate([dst, jnp.full((E_PAD - E,), N, jnp.int32)])
    erT_pad = jnp.pad(erT, ((0, 0), (0, NP - N))).reshape(NC, HH, NP)
    cs = jnp.broadcast_to(C[:, None], (H, LANES)).reshape(NC, HH, LANES)
    zeros4 = jnp.zeros((HH, NP), jnp.float32)
    rowidx = jnp.arange(HH, dtype=jnp.int32)

    aT_full = _sc_softmax(elT_pad, dst_pad, erT_pad, cs, zeros4, rowidx)
    aT = aT_full.reshape(H, E_PAD)[:, :E]

    msg_a = _msg_call(em, aT, W_enc_l, EH_A, 0)
    out_a = _SC_AGG_A(msg_a, dst[:EH_A])
    msg_b = _msg_call(em, aT, W_enc_l, EH_B, EH_A // EB)
    out = _SC_AGG_B(msg_b, dst[EH_A:], out_a)
    return out.reshape(N, H, D_OUT)
